# Initial kernel scaffold; baseline (speedup 1.0000x reference)
import jax
import jax.numpy as jnp
from jax.experimental import pallas as pl


def kernel(stu_emb, t1_prob, t2_prob, classifier_weight):
    raise NotImplementedError("write your pallas kernel here")


# speedup vs baseline: 10.0617x; 10.0617x over previous
import functools

import jax
import jax.numpy as jnp
from jax import lax
from jax.experimental import pallas as pl
from jax.experimental.pallas import tpu as pltpu

N = 16384
D = 768
C = 5
EPS = 1e-6
BLOCK = 1024


def _ranks(p):
    r = jnp.zeros_like(p)
    jidx = lax.broadcasted_iota(jnp.int32, p.shape, 0)
    for k in range(C):
        pk = p[k:k + 1, :]
        beat = (pk > p) | ((pk == p) & (k < jidx))
        r = r + beat.astype(jnp.float32)
    return r


def _tc_body(stu_ref, t1_ref, t2_ref, cw_ref, out_ref):
    i = pl.program_id(0)
    stu = stu_ref[...]
    cw = cw_ref[...]
    t1 = t1_ref[...]
    t2 = t2_ref[...]

    dots = lax.dot_general(cw, stu, (((1,), (1,)), ((), ())),
                           preferred_element_type=jnp.float32,
                           precision=lax.Precision.HIGHEST)
    tt = stu * stu + (2.0 * EPS) * stu
    ones = jnp.ones((1, D), jnp.float32)
    msum = lax.dot_general(ones, tt, (((1,), (1,)), ((), ())),
                           preferred_element_type=jnp.float32,
                           precision=lax.Precision.HIGHEST)
    cvec = (jnp.sum(cw * cw - (2.0 * EPS) * cw, axis=1, keepdims=True)
            + D * EPS * EPS)
    d2 = msum - 2.0 * dots + cvec
    d = jnp.sqrt(jnp.maximum(d2, 0.0))
    dlo = d[0:C, :]
    dhi = d[C:2 * C, :]

    r1 = _ranks(t1)
    r2 = _ranks(t2)
    w1 = jnp.where(r1 >= 0.5, 1.1 - 0.1 * r1, 0.0)
    w2 = jnp.where(r2 >= 0.5, 1.1 - 0.1 * r2, 0.0)
    a1 = (r1 < 0.5).astype(jnp.float32)
    a2 = (r2 < 0.5).astype(jnp.float32)

    da1 = jnp.sum(a2 * dhi, axis=0, keepdims=True)
    da2 = jnp.sum(a1 * dlo, axis=0, keepdims=True)
    term1 = jnp.sum(w1 * jnp.maximum(da1 - dlo + 1.0, 0.0))
    term2 = jnp.sum(w2 * jnp.maximum(da2 - dhi + 1.0, 0.0))
    part = (term1 + term2) * (1.0 / N)

    @pl.when(i == 0)
    def _():
        out_ref[0, 0] = 0.0

    out_ref[0, 0] += part


@jax.jit
def kernel(stu_emb, t1_prob, t2_prob, classifier_weight):
    t1t = t1_prob.T
    t2t = t2_prob.T
    out = pl.pallas_call(
        _tc_body,
        grid=(N // BLOCK,),
        in_specs=[
            pl.BlockSpec((BLOCK, D), lambda i: (i, 0)),
            pl.BlockSpec((C, BLOCK), lambda i: (0, i)),
            pl.BlockSpec((C, BLOCK), lambda i: (0, i)),
            pl.BlockSpec((2 * C, D), lambda i: (0, 0)),
        ],
        out_specs=pl.BlockSpec((1, 1), lambda i: (0, 0),
                               memory_space=pltpu.SMEM),
        out_shape=jax.ShapeDtypeStruct((1, 1), jnp.float32),
    )(stu_emb, t1t, t2t, classifier_weight)
    return out[0, 0]


# speedup vs baseline: 32.0405x; 3.1844x over previous
import functools

import jax
import jax.numpy as jnp
from jax import lax
from jax.experimental import pallas as pl
from jax.experimental.pallas import tpu as pltpu

N = 16384
D = 768
C = 5
EPS = 1e-6
BLOCK = 1024


def _ranks(p):
    r = jnp.zeros_like(p)
    jidx = lax.broadcasted_iota(jnp.int32, p.shape, 0)
    for k in range(C):
        pk = p[k:k + 1, :]
        beat = (pk > p) | ((pk == p) & (k < jidx))
        r = r + beat.astype(jnp.float32)
    return r


def _tc_body(stu_ref, t1_ref, t2_ref, cw_ref, out_ref, cvec_ref):
    i = pl.program_id(0)
    stu = stu_ref[...]
    cw = cw_ref[...]
    t1 = t1_ref[...]
    t2 = t2_ref[...]

    dots = lax.dot_general(cw, stu, (((1,), (1,)), ((), ())),
                           preferred_element_type=jnp.float32,
                           precision=lax.Precision.DEFAULT)
    tt = stu * (stu + 2.0 * EPS)
    ones = jnp.ones((1, D), jnp.float32)
    msum = lax.dot_general(ones, tt, (((1,), (1,)), ((), ())),
                           preferred_element_type=jnp.float32,
                           precision=lax.Precision.DEFAULT)
    @pl.when(i == 0)
    def _():
        cvec_ref[...] = (jnp.sum(cw * (cw - 2.0 * EPS), axis=1, keepdims=True)
                         + D * EPS * EPS)

    cvec = cvec_ref[...]
    d2 = msum - 2.0 * dots + cvec
    d = jnp.sqrt(jnp.maximum(d2, 0.0))
    dlo = d[0:C, :]
    dhi = d[C:2 * C, :]

    r1 = _ranks(t1)
    r2 = _ranks(t2)
    w1 = jnp.where(r1 >= 0.5, 1.1 - 0.1 * r1, 0.0)
    w2 = jnp.where(r2 >= 0.5, 1.1 - 0.1 * r2, 0.0)
    a1 = (r1 < 0.5).astype(jnp.float32)
    a2 = (r2 < 0.5).astype(jnp.float32)

    da1 = jnp.sum(a2 * dhi, axis=0, keepdims=True)
    da2 = jnp.sum(a1 * dlo, axis=0, keepdims=True)
    term1 = jnp.sum(w1 * jnp.maximum(da1 - dlo + 1.0, 0.0))
    term2 = jnp.sum(w2 * jnp.maximum(da2 - dhi + 1.0, 0.0))
    part = (term1 + term2) * (1.0 / N)

    @pl.when(i == 0)
    def _():
        out_ref[0, 0] = 0.0

    out_ref[0, 0] += part


@jax.jit
def kernel(stu_emb, t1_prob, t2_prob, classifier_weight):
    t1t = t1_prob.T
    t2t = t2_prob.T
    out = pl.pallas_call(
        _tc_body,
        grid=(N // BLOCK,),
        in_specs=[
            pl.BlockSpec((BLOCK, D), lambda i: (i, 0)),
            pl.BlockSpec((C, BLOCK), lambda i: (0, i)),
            pl.BlockSpec((C, BLOCK), lambda i: (0, i)),
            pl.BlockSpec((2 * C, D), lambda i: (0, 0)),
        ],
        out_specs=pl.BlockSpec((1, 1), lambda i: (0, 0),
                               memory_space=pltpu.SMEM),
        out_shape=jax.ShapeDtypeStruct((1, 1), jnp.float32),
        scratch_shapes=[pltpu.VMEM((2 * C, 1), jnp.float32)],
    )(stu_emb, t1t, t2t, classifier_weight)
    return out[0, 0]


# speedup vs baseline: 36.9147x; 1.1521x over previous
import functools

import jax
import jax.numpy as jnp
from jax import lax
from jax.experimental import pallas as pl
from jax.experimental.pallas import tpu as pltpu

N = 16384
D = 768
C = 5
EPS = 1e-6
BLOCK = 2048


def _ranks(p):
    r = jnp.zeros_like(p)
    jidx = lax.broadcasted_iota(jnp.int32, p.shape, 0)
    for k in range(C):
        pk = p[k:k + 1, :]
        beat = (pk > p) | ((pk == p) & (k < jidx))
        r = r + beat.astype(jnp.float32)
    return r


def _tc_body(stu_ref, t1_ref, t2_ref, cw_ref, out_ref, cvec_ref):
    i = pl.program_id(0)
    stu = stu_ref[...]
    cw = cw_ref[...]
    t1 = t1_ref[...]
    t2 = t2_ref[...]

    dots = lax.dot_general(cw, stu, (((1,), (1,)), ((), ())),
                           preferred_element_type=jnp.float32,
                           precision=lax.Precision.DEFAULT)
    tt = stu * (stu + 2.0 * EPS)
    ones = jnp.ones((1, D), jnp.float32)
    msum = lax.dot_general(ones, tt, (((1,), (1,)), ((), ())),
                           preferred_element_type=jnp.float32,
                           precision=lax.Precision.DEFAULT)
    @pl.when(i == 0)
    def _():
        cvec_ref[...] = (jnp.sum(cw * (cw - 2.0 * EPS), axis=1, keepdims=True)
                         + D * EPS * EPS)

    cvec = cvec_ref[...]
    d2 = msum - 2.0 * dots + cvec
    d = jnp.sqrt(jnp.maximum(d2, 0.0))
    dlo = d[0:C, :]
    dhi = d[C:2 * C, :]

    r1 = _ranks(t1)
    r2 = _ranks(t2)
    w1 = jnp.where(r1 >= 0.5, 1.1 - 0.1 * r1, 0.0)
    w2 = jnp.where(r2 >= 0.5, 1.1 - 0.1 * r2, 0.0)
    a1 = (r1 < 0.5).astype(jnp.float32)
    a2 = (r2 < 0.5).astype(jnp.float32)

    da1 = jnp.sum(a2 * dhi, axis=0, keepdims=True)
    da2 = jnp.sum(a1 * dlo, axis=0, keepdims=True)
    term1 = jnp.sum(w1 * jnp.maximum(da1 - dlo + 1.0, 0.0))
    term2 = jnp.sum(w2 * jnp.maximum(da2 - dhi + 1.0, 0.0))
    part = (term1 + term2) * (1.0 / N)

    @pl.when(i == 0)
    def _():
        out_ref[0, 0] = 0.0

    out_ref[0, 0] += part


@jax.jit
def kernel(stu_emb, t1_prob, t2_prob, classifier_weight):
    t1t = t1_prob.T
    t2t = t2_prob.T
    out = pl.pallas_call(
        _tc_body,
        grid=(N // BLOCK,),
        in_specs=[
            pl.BlockSpec((BLOCK, D), lambda i: (i, 0)),
            pl.BlockSpec((C, BLOCK), lambda i: (0, i)),
            pl.BlockSpec((C, BLOCK), lambda i: (0, i)),
            pl.BlockSpec((2 * C, D), lambda i: (0, 0)),
        ],
        out_specs=pl.BlockSpec((1, 1), lambda i: (0, 0),
                               memory_space=pltpu.SMEM),
        out_shape=jax.ShapeDtypeStruct((1, 1), jnp.float32),
        scratch_shapes=[pltpu.VMEM((2 * C, 1), jnp.float32)],
    )(stu_emb, t1t, t2t, classifier_weight)
    return out[0, 0]


# speedup vs baseline: 37.3645x; 1.0122x over previous
import functools

import jax
import jax.numpy as jnp
from jax import lax
from jax.experimental import pallas as pl
from jax.experimental.pallas import tpu as pltpu

N = 16384
D = 768
C = 5
EPS = 1e-6
BLOCK = 4096


def _ranks(p):
    r = jnp.zeros_like(p)
    jidx = lax.broadcasted_iota(jnp.int32, p.shape, 0)
    for k in range(C):
        pk = p[k:k + 1, :]
        beat = (pk > p) | ((pk == p) & (k < jidx))
        r = r + beat.astype(jnp.float32)
    return r


def _tc_body(stu_ref, t1_ref, t2_ref, cw_ref, out_ref, cvec_ref):
    i = pl.program_id(0)
    stu = stu_ref[...]
    cw = cw_ref[...]
    t1 = t1_ref[...]
    t2 = t2_ref[...]

    dots = lax.dot_general(cw, stu, (((1,), (1,)), ((), ())),
                           preferred_element_type=jnp.float32,
                           precision=lax.Precision.DEFAULT)
    tt = stu * (stu + 2.0 * EPS)
    ones = jnp.ones((1, D), jnp.float32)
    msum = lax.dot_general(ones, tt, (((1,), (1,)), ((), ())),
                           preferred_element_type=jnp.float32,
                           precision=lax.Precision.DEFAULT)
    @pl.when(i == 0)
    def _():
        cvec_ref[...] = (jnp.sum(cw * (cw - 2.0 * EPS), axis=1, keepdims=True)
                         + D * EPS * EPS)

    cvec = cvec_ref[...]
    d2 = msum - 2.0 * dots + cvec
    d = jnp.sqrt(jnp.maximum(d2, 0.0))
    dlo = d[0:C, :]
    dhi = d[C:2 * C, :]

    r1 = _ranks(t1)
    r2 = _ranks(t2)
    w1 = jnp.where(r1 >= 0.5, 1.1 - 0.1 * r1, 0.0)
    w2 = jnp.where(r2 >= 0.5, 1.1 - 0.1 * r2, 0.0)
    a1 = (r1 < 0.5).astype(jnp.float32)
    a2 = (r2 < 0.5).astype(jnp.float32)

    da1 = jnp.sum(a2 * dhi, axis=0, keepdims=True)
    da2 = jnp.sum(a1 * dlo, axis=0, keepdims=True)
    term1 = jnp.sum(w1 * jnp.maximum(da1 - dlo + 1.0, 0.0))
    term2 = jnp.sum(w2 * jnp.maximum(da2 - dhi + 1.0, 0.0))
    part = (term1 + term2) * (1.0 / N)

    @pl.when(i == 0)
    def _():
        out_ref[0, 0] = 0.0

    out_ref[0, 0] += part


@jax.jit
def kernel(stu_emb, t1_prob, t2_prob, classifier_weight):
    t1t = t1_prob.T
    t2t = t2_prob.T
    out = pl.pallas_call(
        _tc_body,
        grid=(N // BLOCK,),
        in_specs=[
            pl.BlockSpec((BLOCK, D), lambda i: (i, 0)),
            pl.BlockSpec((C, BLOCK), lambda i: (0, i)),
            pl.BlockSpec((C, BLOCK), lambda i: (0, i)),
            pl.BlockSpec((2 * C, D), lambda i: (0, 0)),
        ],
        out_specs=pl.BlockSpec((1, 1), lambda i: (0, 0),
                               memory_space=pltpu.SMEM),
        out_shape=jax.ShapeDtypeStruct((1, 1), jnp.float32),
        scratch_shapes=[pltpu.VMEM((2 * C, 1), jnp.float32)],
    )(stu_emb, t1t, t2t, classifier_weight)
    return out[0, 0]
